# Initial kernel scaffold; baseline (speedup 1.0000x reference)
import jax
import jax.numpy as jnp
from jax.experimental import pallas as pl


def kernel(input_features, segment_ids):
    raise NotImplementedError("write your pallas kernel here")


# speedup vs baseline: 2.2157x; 2.2157x over previous
import functools

import jax
import jax.numpy as jnp
from jax import lax
from jax.experimental import pallas as pl
from jax.experimental.pallas import tpu as pltpu
from jax.experimental.pallas import tpu_sc as plsc

NSEG = 16
ROWS = 32768
D = 512
LANES = 16
VPR = D // LANES
NC = 2
NS = 16
NW = NC * NS
RPW = ROWS // NW
CHUNK = 128
NCHUNK = RPW // CHUNK


@functools.partial(
    pl.kernel,
    out_type=(
        jax.ShapeDtypeStruct((NW, NSEG * D), jnp.float32),
        jax.ShapeDtypeStruct((NW, NSEG), jnp.float32),
    ),
    mesh=plsc.VectorSubcoreMesh(core_axis_name="c", subcore_axis_name="s"),
    compiler_params=pltpu.CompilerParams(needs_layout_passes=False),
    scratch_types=[
        pltpu.VMEM((CHUNK * D,), jnp.float32),
        pltpu.VMEM((RPW,), jnp.int32),
        pltpu.VMEM((NSEG * D,), jnp.float32),
        pltpu.VMEM((NSEG,), jnp.float32),
        pltpu.VMEM((NSEG,), jnp.int32),
    ],
)
def _sc_segment_sum(x_hbm, seg_hbm, zacc_hbm,
                    sums_hbm, cnt_hbm,
                    buf_v, ids_v, acc_v, cntf_v, cnti_v):
    c = lax.axis_index("c")
    s = lax.axis_index("s")
    w = c * NS + s
    base = w * RPW

    pltpu.sync_copy(seg_hbm.at[pl.ds(base, RPW)], ids_v)
    pltpu.sync_copy(zacc_hbm, acc_v)

    zeros_i = jnp.zeros((LANES,), jnp.int32)
    ones_i = jnp.ones((LANES,), jnp.int32)
    iota = lax.iota(jnp.int32, LANES)

    cnti_v[...] = zeros_i
    for i in range(RPW // LANES):
        v = ids_v[pl.ds(i * LANES, LANES)]
        plsc.addupdate_scatter(cnti_v, [v], ones_i)

    counts = cnti_v[...]
    incl = plsc.cumsum(counts)
    excl = incl - counts

    starts = []
    ends = []
    for g in range(NSEG):
        sel = jnp.where(iota == g, excl, 0)
        sel_e = jnp.where(iota == g, incl, 0)
        starts.append(lax.reduce_max(sel, axes=(0,)))
        ends.append(lax.reduce_max(sel_e, axes=(0,)))

    zeros_f = jnp.zeros((LANES,), jnp.float32)

    def chunk_body(j, _):
        pltpu.sync_copy(
            x_hbm.at[pl.ds((base + j * CHUNK) * D, CHUNK * D)], buf_v)
        row0 = j * CHUNK
        for g in range(NSEG):
            lo = jnp.maximum(starts[g] - row0, 0)
            hi = jnp.minimum(ends[g] - row0, CHUNK)

            @pl.when(hi > lo)
            def _():
                def row_body(r, carry):
                    off = r * D
                    return tuple(
                        carry[k] + buf_v[pl.ds(off + k * LANES, LANES)]
                        for k in range(VPR))

                acc = lax.fori_loop(lo, hi, row_body, (zeros_f,) * VPR)
                for k in range(VPR):
                    dst = pl.ds(g * D + k * LANES, LANES)
                    acc_v[dst] = acc_v[dst] + acc[k]
        return 0

    lax.fori_loop(0, NCHUNK, chunk_body, 0)

    cntf_v[...] = counts.astype(jnp.float32)
    pltpu.sync_copy(acc_v, sums_hbm.at[w])
    pltpu.sync_copy(cntf_v, cnt_hbm.at[w])


def _finish_body(sums_ref, cnt_ref, out_ref):
    total = jnp.sum(sums_ref[...], axis=0)
    cnt = jnp.sum(cnt_ref[...], axis=0)
    out_ref[...] = total / jnp.maximum(cnt[:, None], 1.0)


_finish = pl.pallas_call(
    _finish_body,
    out_shape=jax.ShapeDtypeStruct((NSEG, D), jnp.float32),
)


def kernel(input_features, segment_ids):
    x_flat = input_features.reshape(-1)
    seg = segment_ids.astype(jnp.int32)
    zacc = jnp.zeros((NSEG * D,), jnp.float32)
    sums, cnts = _sc_segment_sum(x_flat, seg, zacc)
    sums = sums.reshape(NW, NSEG, D)
    return _finish(sums, cnts)


# speedup vs baseline: 2.6566x; 1.1990x over previous
import functools

import jax
import jax.numpy as jnp
from jax import lax
from jax.experimental import pallas as pl
from jax.experimental.pallas import tpu as pltpu
from jax.experimental.pallas import tpu_sc as plsc

NSEG = 16
ROWS = 32768
D = 512
LANES = 16
VPR = D // LANES
NC = 2
NS = 16
NW = NC * NS
RPW = ROWS // NW
CHUNK = 64
NCHUNK = RPW // CHUNK


@functools.partial(
    pl.kernel,
    out_type=(
        jax.ShapeDtypeStruct((NW, NSEG * D), jnp.float32),
        jax.ShapeDtypeStruct((NW, NSEG), jnp.float32),
    ),
    mesh=plsc.VectorSubcoreMesh(core_axis_name="c", subcore_axis_name="s"),
    compiler_params=pltpu.CompilerParams(needs_layout_passes=False),
    scratch_types=[
        pltpu.VMEM((CHUNK * D,), jnp.float32),
        pltpu.VMEM((CHUNK * D,), jnp.float32),
        pltpu.VMEM((RPW,), jnp.int32),
        pltpu.VMEM((NSEG * D,), jnp.float32),
        pltpu.VMEM((NSEG,), jnp.float32),
        pltpu.VMEM((NSEG,), jnp.int32),
        pltpu.SemaphoreType.DMA,
        pltpu.SemaphoreType.DMA,
    ],
)
def _sc_segment_sum(x_hbm, seg_hbm, zacc_hbm,
                    sums_hbm, cnt_hbm,
                    buf0_v, buf1_v, ids_v, acc_v, cntf_v, cnti_v,
                    sem0, sem1):
    c = lax.axis_index("c")
    s = lax.axis_index("s")
    w = c * NS + s
    base = w * RPW

    pltpu.sync_copy(seg_hbm.at[pl.ds(base, RPW)], ids_v)
    pltpu.sync_copy(zacc_hbm, acc_v)

    zeros_i = jnp.zeros((LANES,), jnp.int32)
    ones_i = jnp.ones((LANES,), jnp.int32)
    zeros_f = jnp.zeros((LANES,), jnp.float32)
    iota = lax.iota(jnp.int32, LANES)

    cnti_v[...] = zeros_i
    for i in range(RPW // LANES):
        v = ids_v[pl.ds(i * LANES, LANES)]
        plsc.addupdate_scatter(cnti_v, [v], ones_i)

    counts = cnti_v[...]
    incl = plsc.cumsum(counts)
    excl = incl - counts

    def chunk_slice(j):
        return x_hbm.at[pl.ds((base + j * CHUNK) * D, CHUNK * D)]

    def process(row0, buf):
        hi_row = row0 + CHUNK
        c_end = plsc.all_reduce_population_count(incl > row0)
        c_start = plsc.all_reduce_population_count(excl < hi_row)
        g_lo = NSEG - lax.reduce_max(c_end, axes=(0,))
        g_hi = lax.reduce_max(c_start, axes=(0,))

        def seg_body(g, _):
            sel = iota == g
            start_g = lax.reduce_max(jnp.where(sel, excl, 0), axes=(0,))
            end_g = lax.reduce_max(jnp.where(sel, incl, 0), axes=(0,))
            lo = jnp.maximum(start_g - row0, 0)
            hi = jnp.minimum(end_g - row0, CHUNK)
            n = hi - lo
            half = n >> 1

            def row2(r, carry):
                off = (lo + 2 * r) * D
                return tuple(
                    carry[k]
                    + buf[pl.ds(off + k * LANES, LANES)]
                    + buf[pl.ds(off + D + k * LANES, LANES)]
                    for k in range(VPR))

            acc = lax.fori_loop(0, half, row2, (zeros_f,) * VPR)

            last = jnp.maximum(hi - 1, 0) * D
            odd = (n & 1) == 1
            for k in range(VPR):
                x_last = buf[pl.ds(last + k * LANES, LANES)]
                total = acc[k] + jnp.where(odd, x_last, 0.0)
                dst = pl.ds(g * D + k * LANES, LANES)
                acc_v[dst] = acc_v[dst] + total
            return 0

        lax.fori_loop(g_lo, g_hi, seg_body, 0)

    pltpu.async_copy(chunk_slice(0), buf0_v, sem0)

    def body2(t, _):
        j0 = 2 * t
        pltpu.async_copy(chunk_slice(j0 + 1), buf1_v, sem1)
        pltpu.make_async_copy(chunk_slice(j0), buf0_v, sem0).wait()
        process(j0 * CHUNK, buf0_v)

        @pl.when(t < NCHUNK // 2 - 1)
        def _():
            pltpu.async_copy(chunk_slice(j0 + 2), buf0_v, sem0)

        pltpu.make_async_copy(chunk_slice(j0 + 1), buf1_v, sem1).wait()
        process((j0 + 1) * CHUNK, buf1_v)
        return 0

    lax.fori_loop(0, NCHUNK // 2, body2, 0)

    cntf_v[...] = counts.astype(jnp.float32)
    pltpu.sync_copy(acc_v, sums_hbm.at[w])
    pltpu.sync_copy(cntf_v, cnt_hbm.at[w])


def _finish_body(sums_ref, cnt_ref, out_ref):
    total = jnp.sum(sums_ref[...], axis=0)
    cnt = jnp.sum(cnt_ref[...], axis=0)
    out_ref[...] = total / jnp.maximum(cnt[:, None], 1.0)


_finish = pl.pallas_call(
    _finish_body,
    out_shape=jax.ShapeDtypeStruct((NSEG, D), jnp.float32),
)


def kernel(input_features, segment_ids):
    x_flat = input_features.reshape(-1)
    seg = segment_ids.astype(jnp.int32)
    zacc = jnp.zeros((NSEG * D,), jnp.float32)
    sums, cnts = _sc_segment_sum(x_flat, seg, zacc)
    sums = sums.reshape(NW, NSEG, D)
    return _finish(sums, cnts)


# speedup vs baseline: 4.8633x; 1.8306x over previous
import functools

import jax
import jax.numpy as jnp
from jax import lax
from jax.experimental import pallas as pl
from jax.experimental.pallas import tpu as pltpu
from jax.experimental.pallas import tpu_sc as plsc

NSEG = 16
ROWS = 32768
D = 512
LANES = 16
VPR = D // LANES
NC = 2
NS = 16
NW = NC * NS
RPW = ROWS // NW
CHUNK = 64
NCHUNK = RPW // CHUNK


@functools.partial(
    pl.kernel,
    out_type=(
        jax.ShapeDtypeStruct((NW, NSEG * D), jnp.float32),
        jax.ShapeDtypeStruct((NW, NSEG), jnp.float32),
    ),
    mesh=plsc.VectorSubcoreMesh(core_axis_name="c", subcore_axis_name="s"),
    compiler_params=pltpu.CompilerParams(needs_layout_passes=False),
    scratch_types=[
        pltpu.VMEM((CHUNK, D), jnp.float32),
        pltpu.VMEM((CHUNK, D), jnp.float32),
        pltpu.VMEM((RPW,), jnp.int32),
        pltpu.VMEM((NSEG * D,), jnp.float32),
        pltpu.VMEM((NSEG,), jnp.float32),
        pltpu.VMEM((NSEG,), jnp.int32),
        pltpu.SemaphoreType.DMA,
        pltpu.SemaphoreType.DMA,
    ],
)
def _sc_segment_sum(x_hbm, seg_hbm, zacc_hbm,
                    sums_hbm, cnt_hbm,
                    buf0_v, buf1_v, ids_v, acc_v, cntf_v, cnti_v,
                    sem0, sem1):
    c = lax.axis_index("c")
    s = lax.axis_index("s")
    w = c * NS + s
    base = w * RPW

    pltpu.sync_copy(seg_hbm.at[pl.ds(base, RPW)], ids_v)
    pltpu.sync_copy(zacc_hbm, acc_v)

    zeros_i = jnp.zeros((LANES,), jnp.int32)
    ones_i = jnp.ones((LANES,), jnp.int32)
    zeros_f = jnp.zeros((LANES,), jnp.float32)
    iota = lax.iota(jnp.int32, LANES)

    cnti_v[...] = zeros_i
    for i in range(RPW // LANES):
        v = ids_v[pl.ds(i * LANES, LANES)]
        plsc.addupdate_scatter(cnti_v, [v], ones_i)

    counts = cnti_v[...]
    incl = plsc.cumsum(counts)
    excl = incl - counts

    def chunk_slice(j):
        return x_hbm.at[pl.ds(base + j * CHUNK, CHUNK)]

    def process(row0, buf):
        hi_row = row0 + CHUNK
        c_end = plsc.all_reduce_population_count(incl > row0)
        c_start = plsc.all_reduce_population_count(excl < hi_row)
        g_lo = NSEG - lax.reduce_max(c_end, axes=(0,))
        g_hi = lax.reduce_max(c_start, axes=(0,))

        def seg_body(g, _):
            sel = iota == g
            start_g = lax.reduce_max(jnp.where(sel, excl, 0), axes=(0,))
            end_g = lax.reduce_max(jnp.where(sel, incl, 0), axes=(0,))
            lo = jnp.maximum(start_g - row0, 0)
            hi = jnp.minimum(end_g - row0, CHUNK)
            n = hi - lo
            half = n >> 1

            def row2(r, carry):
                r0 = lo + 2 * r
                return tuple(
                    carry[k]
                    + buf[r0, pl.ds(k * LANES, LANES)]
                    + buf[r0 + 1, pl.ds(k * LANES, LANES)]
                    for k in range(VPR))

            acc = lax.fori_loop(0, half, row2, (zeros_f,) * VPR)

            last = jnp.maximum(hi - 1, 0)
            odd = (n & 1) == 1
            for k in range(VPR):
                x_last = buf[last, pl.ds(k * LANES, LANES)]
                total = acc[k] + jnp.where(odd, x_last, 0.0)
                dst = pl.ds(g * D + k * LANES, LANES)
                acc_v[dst] = acc_v[dst] + total
            return 0

        lax.fori_loop(g_lo, g_hi, seg_body, 0)

    pltpu.async_copy(chunk_slice(0), buf0_v, sem0)

    def body2(t, _):
        j0 = 2 * t
        pltpu.async_copy(chunk_slice(j0 + 1), buf1_v, sem1)
        pltpu.make_async_copy(chunk_slice(j0), buf0_v, sem0).wait()
        process(j0 * CHUNK, buf0_v)

        @pl.when(t < NCHUNK // 2 - 1)
        def _():
            pltpu.async_copy(chunk_slice(j0 + 2), buf0_v, sem0)

        pltpu.make_async_copy(chunk_slice(j0 + 1), buf1_v, sem1).wait()
        process((j0 + 1) * CHUNK, buf1_v)
        return 0

    lax.fori_loop(0, NCHUNK // 2, body2, 0)

    cntf_v[...] = counts.astype(jnp.float32)
    pltpu.sync_copy(acc_v, sums_hbm.at[w])
    pltpu.sync_copy(cntf_v, cnt_hbm.at[w])


def _finish_body(sums_ref, cnt_ref, out_ref):
    total = jnp.sum(sums_ref[...], axis=0)
    cnt = jnp.sum(cnt_ref[...], axis=0)
    out_ref[...] = total / jnp.maximum(cnt[:, None], 1.0)


_finish = pl.pallas_call(
    _finish_body,
    out_shape=jax.ShapeDtypeStruct((NSEG, D), jnp.float32),
)


def kernel(input_features, segment_ids):
    seg = segment_ids.astype(jnp.int32)
    zacc = jnp.zeros((NSEG * D,), jnp.float32)
    sums, cnts = _sc_segment_sum(input_features, seg, zacc)
    sums = sums.reshape(NW, NSEG, D)
    return _finish(sums, cnts)
